# Initial kernel scaffold; baseline (speedup 1.0000x reference)
import jax
import jax.numpy as jnp
from jax.experimental import pallas as pl


def kernel(x, table):
    raise NotImplementedError("write your pallas kernel here")


# speedup vs baseline: 7.3440x; 7.3440x over previous
import functools

import jax
import jax.numpy as jnp
from jax import lax
from jax.experimental import pallas as pl
from jax.experimental.pallas import tpu as pltpu
from jax.experimental.pallas import tpu_sc as plsc

_IDX_LANES = 128
_CHUNK = 256


def _build_gather(V, D, B, NC, NS):
    NW = NC * NS
    b_per_w = B // NW
    group = 8 * _IDX_LANES
    n_groups = b_per_w // group
    subs_per_group = group // _CHUNK
    rows_per_chunk = _CHUNK // _IDX_LANES

    mesh = plsc.VectorSubcoreMesh(core_axis_name="c", subcore_axis_name="s")

    @functools.partial(
        pl.kernel,
        mesh=mesh,
        out_type=jax.ShapeDtypeStruct((B, D), jnp.float32),
        scratch_types=[
            pltpu.VMEM((8, _IDX_LANES), jnp.int32),
            pltpu.VMEM((_CHUNK, D), jnp.float32),
            pltpu.SemaphoreType.DMA,
        ],
    )
    def gather_kernel(table_hbm, idx_hbm, out_hbm, idx_v, rows_v, sem):
        wid = lax.axis_index("s") * NC + lax.axis_index("c")
        base = wid * b_per_w

        def body(g, carry):
            g_off = pl.multiple_of(base + g * group, group)
            idx_row = pl.multiple_of(g_off // _IDX_LANES, 8)
            pltpu.sync_copy(idx_hbm.at[pl.ds(idx_row, 8)], idx_v)
            for sub in range(subs_per_group):
                copies = [
                    pltpu.async_copy(
                        table_hbm.at[idx_v.at[sub * rows_per_chunk + j]],
                        rows_v.at[pl.ds(j * _IDX_LANES, _IDX_LANES)],
                        sem,
                    )
                    for j in range(rows_per_chunk)
                ]
                for c in copies:
                    c.wait()
                pltpu.sync_copy(
                    rows_v,
                    out_hbm.at[
                        pl.ds(pl.multiple_of(g_off + sub * _CHUNK, _CHUNK), _CHUNK)
                    ],
                )
            return carry

        lax.fori_loop(0, n_groups, body, 0)

    return gather_kernel


def kernel(x, table):
    V, D = table.shape
    B = x.size
    info = plsc.get_sparse_core_info()
    NC, NS = info.num_cores, info.num_subcores
    assert B % (NC * NS) == 0 and (B // (NC * NS)) % (8 * _IDX_LANES) == 0
    xf = x.reshape(B // _IDX_LANES, _IDX_LANES).astype(jnp.int32)
    out = _build_gather(V, D, B, NC, NS)(table, xf)
    return out.reshape(*x.shape, D)


# speedup vs baseline: 9.1480x; 1.2456x over previous
import functools

import jax
import jax.numpy as jnp
from jax import lax
from jax.experimental import pallas as pl
from jax.experimental.pallas import tpu as pltpu
from jax.experimental.pallas import tpu_sc as plsc

_CH = 128
_NSLOT = 4


def _build_gather(V, D, B, NC, NS):
    NW = NC * NS
    b_per_w = B // NW
    n_rows = b_per_w // _CH
    n_grp = n_rows // _NSLOT

    mesh = plsc.VectorSubcoreMesh(core_axis_name="c", subcore_axis_name="s")

    @functools.partial(
        pl.kernel,
        mesh=mesh,
        out_type=jax.ShapeDtypeStruct((B, D), jnp.float32),
        scratch_types=[
            pltpu.VMEM((n_rows, _CH), jnp.int32),
            pltpu.VMEM((_NSLOT, _CH, D), jnp.float32),
            pltpu.SemaphoreType.DMA,
            pltpu.SemaphoreType.DMA,
            pltpu.SemaphoreType.DMA,
            pltpu.SemaphoreType.DMA,
            pltpu.SemaphoreType.DMA,
            pltpu.SemaphoreType.DMA,
            pltpu.SemaphoreType.DMA,
            pltpu.SemaphoreType.DMA,
        ],
    )
    def gather_kernel(table_hbm, idx_hbm, out_hbm, idx_v, rows_v, *sems):
        gsem = sems[:_NSLOT]
        wsem = sems[_NSLOT:]
        wid = lax.axis_index("s") * NC + lax.axis_index("c")
        base = pl.multiple_of(wid * b_per_w, b_per_w)
        idx_base = pl.multiple_of(base // _CH, n_rows)

        def fire_gather(c, slot):
            pltpu.async_copy(
                table_hbm.at[idx_v.at[c]],
                rows_v.at[slot],
                gsem[slot],
            )

        def wait_gather(slot):
            pltpu.make_async_copy(
                table_hbm.at[pl.ds(0, _CH)], rows_v.at[slot], gsem[slot]
            ).wait()

        def fire_write(c, slot):
            off = pl.multiple_of(base + c * _CH, _CH)
            pltpu.async_copy(rows_v.at[slot], out_hbm.at[pl.ds(off, _CH)],
                             wsem[slot])

        def wait_write(slot):
            pltpu.make_async_copy(
                rows_v.at[slot], out_hbm.at[pl.ds(0, _CH)], wsem[slot]
            ).wait()

        pltpu.sync_copy(idx_hbm.at[pl.ds(idx_base, n_rows)], idx_v)

        fire_gather(0, 0)
        fire_gather(1, 1)
        wait_gather(0); fire_write(0, 0); fire_gather(2, 2)
        wait_gather(1); fire_write(1, 1); fire_gather(3, 3)
        wait_gather(2); fire_write(2, 2); wait_write(0); fire_gather(4, 0)
        wait_gather(3); fire_write(3, 3); wait_write(1); fire_gather(5, 1)

        def body(g, carry):
            c0 = g * _NSLOT
            for k in range(_NSLOT):
                slot = k
                nslot = (k + 2) % _NSLOT
                wait_gather(slot)
                fire_write(c0 + k, slot)
                wait_write(nslot)
                fire_gather(c0 + k + 2, nslot)
            return carry

        lax.fori_loop(1, n_grp - 1, body, 0)

        c0 = (n_grp - 1) * _NSLOT
        wait_gather(0); fire_write(c0 + 0, 0); wait_write(2); fire_gather(c0 + 2, 2)
        wait_gather(1); fire_write(c0 + 1, 1); wait_write(3); fire_gather(c0 + 3, 3)
        wait_gather(2); fire_write(c0 + 2, 2)
        wait_gather(3); fire_write(c0 + 3, 3)
        wait_write(0); wait_write(1); wait_write(2); wait_write(3)

    return gather_kernel


def kernel(x, table):
    V, D = table.shape
    B = x.size
    info = plsc.get_sparse_core_info()
    NC, NS = info.num_cores, info.num_subcores
    per_w = B // (NC * NS)
    assert B % (NC * NS) == 0 and per_w % (_CH * _NSLOT) == 0
    assert per_w // _CH >= 3 * _NSLOT
    xf = x.reshape(B // _CH, _CH).astype(jnp.int32)
    out = _build_gather(V, D, B, NC, NS)(table, xf)
    return out.reshape(*x.shape, D)


# speedup vs baseline: 9.1773x; 1.0032x over previous
import functools

import jax
import jax.numpy as jnp
from jax import lax
from jax.experimental import pallas as pl
from jax.experimental.pallas import tpu as pltpu
from jax.experimental.pallas import tpu_sc as plsc

_IDXW = 128
_CH = 64
_NSLOT = 8
_DIST = 4


def _build_gather(V, D, B, NC, NS):
    NW = NC * NS
    b_per_w = B // NW
    n_rows = b_per_w // _IDXW
    n_ch = b_per_w // _CH
    n_grp = n_ch // _NSLOT

    mesh = plsc.VectorSubcoreMesh(core_axis_name="c", subcore_axis_name="s")

    @functools.partial(
        pl.kernel,
        mesh=mesh,
        out_type=jax.ShapeDtypeStruct((B, D), jnp.float32),
        scratch_types=[
            pltpu.VMEM((n_rows, _IDXW), jnp.int32),
            pltpu.VMEM((_NSLOT, _CH, D), jnp.float32),
        ]
        + [pltpu.SemaphoreType.DMA] * (2 * _NSLOT),
    )
    def gather_kernel(table_hbm, idx_hbm, out_hbm, idx_v, rows_v, *sems):
        gsem = sems[:_NSLOT]
        wsem = sems[_NSLOT:]
        wid = lax.axis_index("s") * NC + lax.axis_index("c")
        base = pl.multiple_of(wid * b_per_w, b_per_w)
        idx_base = pl.multiple_of(base // _IDXW, n_rows)

        def fire_gather(c, slot, half):
            pltpu.async_copy(
                table_hbm.at[idx_v.at[c // 2, pl.ds(half * _CH, _CH)]],
                rows_v.at[slot],
                gsem[slot],
            )

        def wait_gather(slot):
            pltpu.make_async_copy(
                table_hbm.at[pl.ds(0, _CH)], rows_v.at[slot], gsem[slot]
            ).wait()

        def fire_write(c, slot):
            off = pl.multiple_of(base + c * _CH, _CH)
            pltpu.async_copy(rows_v.at[slot], out_hbm.at[pl.ds(off, _CH)],
                             wsem[slot])

        def wait_write(slot):
            pltpu.make_async_copy(
                rows_v.at[slot], out_hbm.at[pl.ds(0, _CH)], wsem[slot]
            ).wait()

        pltpu.sync_copy(idx_hbm.at[pl.ds(idx_base, n_rows)], idx_v)

        for k in range(_DIST):
            fire_gather(k, k, k & 1)
        for k in range(_NSLOT):
            wait_gather(k)
            fire_write(k, k)
            if k + _DIST < _NSLOT:
                fire_gather(k + _DIST, k + _DIST, (k + _DIST) & 1)
            else:
                wait_write((k + _DIST) % _NSLOT)
                fire_gather(k + _DIST, (k + _DIST) % _NSLOT, (k + _DIST) & 1)

        def body(g, carry):
            c0 = g * _NSLOT
            for k in range(_NSLOT):
                nslot = (k + _DIST) % _NSLOT
                wait_gather(k)
                fire_write(c0 + k, k)
                wait_write(nslot)
                fire_gather(c0 + k + _DIST, nslot, (k + _DIST) & 1)
            return carry

        lax.fori_loop(1, n_grp - 1, body, 0)

        c0 = (n_grp - 1) * _NSLOT
        for k in range(_NSLOT):
            nslot = (k + _DIST) % _NSLOT
            wait_gather(k)
            fire_write(c0 + k, k)
            if k + _DIST < _NSLOT:
                wait_write(nslot)
                fire_gather(c0 + k + _DIST, nslot, (k + _DIST) & 1)
        for k in range(_NSLOT):
            wait_write(k)

    return gather_kernel


def kernel(x, table):
    V, D = table.shape
    B = x.size
    info = plsc.get_sparse_core_info()
    NC, NS = info.num_cores, info.num_subcores
    per_w = B // (NC * NS)
    assert B % (NC * NS) == 0 and per_w % (_CH * _NSLOT) == 0
    assert per_w % _IDXW == 0 and per_w // _CH >= 3 * _NSLOT
    xf = x.reshape(B // _IDXW, _IDXW).astype(jnp.int32)
    out = _build_gather(V, D, B, NC, NS)(table, xf)
    return out.reshape(*x.shape, D)
